# baseline (speedup 1.0000x reference)
import functools

import jax
import jax.numpy as jnp
import numpy as np
from jax import lax
from jax.experimental import pallas as pl
from jax.experimental.pallas import tpu as pltpu
from jax.experimental.pallas import tpu_sc as plsc

_B, _S, _K, _D = 1024, 16, 64, 32
_ROWS = _B * _S
_R = 512
_BB = 512


def _sample_body(bits_ref, lp_ref, out_ref):
    bits = bits_ref[...]
    fb = (bits >> jnp.uint32(9)) | jnp.uint32(0x3F800000)
    f = lax.bitcast_convert_type(fb, jnp.float32) - jnp.float32(1.0)
    tiny = jnp.float32(np.finfo(np.float32).tiny)
    u = jnp.maximum(tiny, f * (jnp.float32(1.0) - tiny) + tiny)
    g = -jnp.log(-jnp.log(u))
    s = g + lp_ref[...]
    m = jnp.max(s, axis=1, keepdims=True)
    ik = lax.broadcasted_iota(jnp.int32, (_R, _K), 1)
    out_ref[...] = jnp.min(jnp.where(s == m, ik, jnp.int32(_K)),
                           axis=1, keepdims=True)


def _select_body(mu_ref, sg_ref, z_ref, omu_ref, osg_ref):
    zrow = z_ref[0]
    acc_mu = mu_ref[0, 0]
    acc_sg = sg_ref[0, 0]
    for k in range(1, _K):
        mask = zrow == jnp.int32(k)
        acc_mu = jnp.where(mask, mu_ref[0, k], acc_mu)
        acc_sg = jnp.where(mask, sg_ref[0, k], acc_sg)
    omu_ref[0] = acc_mu
    osg_ref[0] = acc_sg


def _select_body_one(mu_ref, z_ref, omu_ref):
    zrow = z_ref[0]
    acc = mu_ref[0, 0]
    for k in range(1, _K):
        acc = jnp.where(zrow == jnp.int32(k), mu_ref[0, k], acc)
    omu_ref[0] = acc


_NW = 32
_TPW = (_S * 4 * 8) // _NW


def _sc_select(sig_hbm, z_hbm, out_hbm, stage_v, z_v, out_v):
    wid = lax.axis_index("s") * 2 + lax.axis_index("c")

    def task(i, carry):
        t = wid * _TPW + i
        s = t // 32
        dhi = (t % 32) // 8
        bhi = t % 8
        pltpu.sync_copy(z_hbm.at[pl.ds(s, 1), pl.ds(bhi * 128, 128)], z_v)
        pltpu.sync_copy(
            sig_hbm.at[s, :, pl.ds(dhi * 8, 8), pl.ds(bhi * 128, 128)],
            stage_v)
        for g in range(8):
            zv = z_v[0, pl.ds(g * 16, 16)]
            col = lax.iota(jnp.int32, 16) + jnp.int32(g * 16)
            for dlo in range(8):
                row_d = jnp.full((16,), dlo, jnp.int32)
                out_v[dlo, pl.ds(g * 16, 16)] = plsc.load_gather(
                    stage_v, [zv, row_d, col])
        pltpu.sync_copy(
            out_v, out_hbm.at[s, pl.ds(dhi * 8, 8), pl.ds(bhi * 128, 128)])
        return carry

    lax.fori_loop(0, _TPW, task, 0)


def _sc_select_call():
    return functools.partial(
        pl.kernel,
        out_type=jax.ShapeDtypeStruct((_S, _D, _B), jnp.float32),
        mesh=plsc.VectorSubcoreMesh(core_axis_name="c", subcore_axis_name="s"),
        scratch_types=[pltpu.VMEM((_K, 8, 128), jnp.float32),
                       pltpu.VMEM((1, 128), jnp.int32),
                       pltpu.VMEM((8, 128), jnp.float32)],
        compiler_params=pltpu.CompilerParams(needs_layout_passes=False),
    )


def kernel(mus, sigmas, pi):
    zkey = jax.random.fold_in(jax.random.key(0), 123)
    bits = jax.random.bits(zkey, (_B, _S, _K), jnp.uint32).reshape(_ROWS, _K)
    lp_block = jnp.tile(jnp.log(pi), (_R // _S, 1))

    z_flat = pl.pallas_call(
        _sample_body,
        grid=(_ROWS // _R,),
        in_specs=[
            pl.BlockSpec((_R, _K), lambda i: (i, 0)),
            pl.BlockSpec((_R, _K), lambda i: (0, 0)),
        ],
        out_specs=pl.BlockSpec((_R, 1), lambda i: (i, 0)),
        out_shape=jax.ShapeDtypeStruct((_ROWS, 1), jnp.int32),
    )(bits, lp_block)
    z_sb = z_flat.reshape(_B, _S).T.reshape(_S, 1, _B)

    mus_t = mus.transpose(1, 2, 3, 0)
    sig_t = sigmas.transpose(1, 2, 3, 0)

    osg_t = _sc_select_call()(_sc_select)(sig_t, z_sb.reshape(_S, _B))

    omu_t = pl.pallas_call(
        _select_body_one,
        grid=(_S, _B // _BB),
        in_specs=[
            pl.BlockSpec((1, _K, _D, _BB), lambda s, b: (s, 0, 0, b)),
            pl.BlockSpec((1, 1, _BB), lambda s, b: (s, 0, b)),
        ],
        out_specs=pl.BlockSpec((1, _D, _BB), lambda s, b: (s, 0, b)),
        out_shape=jax.ShapeDtypeStruct((_S, _D, _B), jnp.float32),
    )(mus_t, z_sb)

    return omu_t.transpose(2, 0, 1), osg_t.transpose(2, 0, 1)


# speedup vs baseline: 1.0885x; 1.0885x over previous
import functools

import jax
import jax.numpy as jnp
import numpy as np
from jax import lax
from jax.experimental import pallas as pl
from jax.experimental.pallas import tpu as pltpu
from jax.experimental.pallas import tpu_sc as plsc

_B, _S, _K, _D = 1024, 16, 64, 32
_ROWS = _B * _S
_R = 512
_BB = 512
_S_SC = 13
_S_TC = _S - _S_SC


def _rotl(x, d):
    return (x << jnp.uint32(d)) | (x >> jnp.uint32(32 - d))


def _sample_body(key_ref, lp_ref, out_ref):
    k1 = key_ref[0, 0]
    k2 = key_ref[0, 1]
    ks2 = k1 ^ k2 ^ jnp.uint32(0x1BD11BDA)
    r0 = jnp.uint32(pl.program_id(0) * _R)
    jrow = lax.broadcasted_iota(jnp.uint32, (_R, _K), 0)
    jcol = lax.broadcasted_iota(jnp.uint32, (_R, _K), 1)
    j = (r0 + jrow) * jnp.uint32(_K) + jcol
    x1 = jnp.zeros((_R, _K), jnp.uint32) + k1
    x2 = j + k2
    rot_a = (13, 15, 26, 6)
    rot_b = (17, 29, 16, 24)

    def group(x1, x2, rots):
        for r in rots:
            x1 = x1 + x2
            x2 = _rotl(x2, r)
            x2 = x1 ^ x2
        return x1, x2

    x1, x2 = group(x1, x2, rot_a)
    x1, x2 = x1 + k2, x2 + ks2 + jnp.uint32(1)
    x1, x2 = group(x1, x2, rot_b)
    x1, x2 = x1 + ks2, x2 + k1 + jnp.uint32(2)
    x1, x2 = group(x1, x2, rot_a)
    x1, x2 = x1 + k1, x2 + k2 + jnp.uint32(3)
    x1, x2 = group(x1, x2, rot_b)
    x1, x2 = x1 + k2, x2 + ks2 + jnp.uint32(4)
    x1, x2 = group(x1, x2, rot_a)
    x1, x2 = x1 + ks2, x2 + k1 + jnp.uint32(5)
    bits = x1 ^ x2
    fb = (bits >> jnp.uint32(9)) | jnp.uint32(0x3F800000)
    f = lax.bitcast_convert_type(fb, jnp.float32) - jnp.float32(1.0)
    tiny = jnp.float32(np.finfo(np.float32).tiny)
    u = jnp.maximum(tiny, f * (jnp.float32(1.0) - tiny) + tiny)
    g = -jnp.log(-jnp.log(u))
    s = g + lp_ref[...]
    m = jnp.max(s, axis=1, keepdims=True)
    ik = lax.broadcasted_iota(jnp.int32, (_R, _K), 1)
    out_ref[...] = jnp.min(jnp.where(s == m, ik, jnp.int32(_K)),
                           axis=1, keepdims=True)


def _select_body_one(mu_ref, z_ref, omu_ref):
    zrow = z_ref[0]
    acc = mu_ref[0, 0]
    for k in range(1, _K):
        acc = jnp.where(zrow == jnp.int32(k), mu_ref[0, k], acc)
    omu_ref[0] = acc


_NW = 32
_TPW = (_S_SC * 4 * 8) // _NW


def _sc_select(sig_hbm, z_hbm, osg_hbm, stage_v, z_v, out_v):
    wid = lax.axis_index("s") * 2 + lax.axis_index("c")

    def task(i, carry):
        t = wid * _TPW + i
        s = t // 32
        dhi = (t % 32) // 8
        bhi = t % 8
        pltpu.sync_copy(z_hbm.at[pl.ds(s, 1), pl.ds(bhi * 128, 128)], z_v)
        pltpu.sync_copy(
            sig_hbm.at[s, :, pl.ds(dhi * 8, 8), pl.ds(bhi * 128, 128)],
            stage_v)
        for g in range(8):
            zv = z_v[0, pl.ds(g * 16, 16)]
            col = lax.iota(jnp.int32, 16) + jnp.int32(g * 16)
            for dlo in range(8):
                row_d = jnp.full((16,), dlo, jnp.int32)
                out_v[dlo, pl.ds(g * 16, 16)] = plsc.load_gather(
                    stage_v, [zv, row_d, col])
        pltpu.sync_copy(
            out_v, osg_hbm.at[s, pl.ds(dhi * 8, 8), pl.ds(bhi * 128, 128)])
        return carry

    lax.fori_loop(0, _TPW, task, 0)


def _sc_select_call():
    return functools.partial(
        pl.kernel,
        out_type=jax.ShapeDtypeStruct((_S_SC, _D, _B), jnp.float32),
        mesh=plsc.VectorSubcoreMesh(core_axis_name="c", subcore_axis_name="s"),
        scratch_types=[pltpu.VMEM((_K, 8, 128), jnp.float32),
                       pltpu.VMEM((1, 128), jnp.int32),
                       pltpu.VMEM((8, 128), jnp.float32)],
        compiler_params=pltpu.CompilerParams(needs_layout_passes=False),
    )


def kernel(mus, sigmas, pi):
    zkey = jax.random.fold_in(jax.random.key(0), 123)
    kd = jax.random.key_data(zkey).astype(jnp.uint32).reshape(1, 2)
    lp_block = jnp.tile(jnp.log(pi), (_R // _S, 1))

    z_flat = pl.pallas_call(
        _sample_body,
        grid=(_ROWS // _R,),
        in_specs=[
            pl.BlockSpec((1, 2), lambda i: (0, 0)),
            pl.BlockSpec((_R, _K), lambda i: (0, 0)),
        ],
        out_specs=pl.BlockSpec((_R, 1), lambda i: (i, 0)),
        out_shape=jax.ShapeDtypeStruct((_ROWS, 1), jnp.int32),
    )(kd, lp_block)
    z_sb = z_flat.reshape(_B, _S).T.reshape(_S, 1, _B)

    mus_t = mus.transpose(1, 2, 3, 0)
    sig_t = sigmas.transpose(1, 2, 3, 0)

    osg_lo = _sc_select_call()(_sc_select)(sig_t, z_sb.reshape(_S, _B))

    omu_t = pl.pallas_call(
        _select_body_one,
        grid=(_S, _B // _BB),
        in_specs=[
            pl.BlockSpec((1, _K, _D, _BB), lambda s, b: (s, 0, 0, b)),
            pl.BlockSpec((1, 1, _BB), lambda s, b: (s, 0, b)),
        ],
        out_specs=pl.BlockSpec((1, _D, _BB), lambda s, b: (s, 0, b)),
        out_shape=jax.ShapeDtypeStruct((_S, _D, _B), jnp.float32),
    )(mus_t, z_sb)

    osg_hi = pl.pallas_call(
        _select_body_one,
        grid=(_S_TC, _B // _BB),
        in_specs=[
            pl.BlockSpec((1, _K, _D, _BB), lambda s, b: (s + _S_SC, 0, 0, b)),
            pl.BlockSpec((1, 1, _BB), lambda s, b: (s + _S_SC, 0, b)),
        ],
        out_specs=pl.BlockSpec((1, _D, _BB), lambda s, b: (s, 0, b)),
        out_shape=jax.ShapeDtypeStruct((_S_TC, _D, _B), jnp.float32),
    )(sig_t, z_sb)

    osg_t = jnp.concatenate([osg_lo, osg_hi], axis=0)
    return omu_t.transpose(2, 0, 1), osg_t.transpose(2, 0, 1)


# speedup vs baseline: 1.2197x; 1.1205x over previous
import functools

import jax
import jax.numpy as jnp
import numpy as np
from jax import lax
from jax.experimental import pallas as pl
from jax.experimental.pallas import tpu as pltpu
from jax.experimental.pallas import tpu_sc as plsc

_B, _S, _K, _D = 1024, 16, 64, 32
_ROWS = _B * _S
_R = 512
_BB = 512
_S_SC = 13
_S_TC = _S - _S_SC


def _rotl(x, d):
    return (x << jnp.uint32(d)) | (x >> jnp.uint32(32 - d))


def _sample_body(key_ref, lp_ref, out_ref):
    k1 = key_ref[0, 0]
    k2 = key_ref[0, 1]
    ks2 = k1 ^ k2 ^ jnp.uint32(0x1BD11BDA)
    _R2, _K2 = _R // 2, 2 * _K
    j0 = jnp.uint32(pl.program_id(0) * (_R * _K))
    jrow = lax.broadcasted_iota(jnp.uint32, (_R2, _K2), 0)
    jcol = lax.broadcasted_iota(jnp.uint32, (_R2, _K2), 1)
    j = j0 + jrow * jnp.uint32(_K2) + jcol
    x1 = jnp.zeros((_R2, _K2), jnp.uint32) + k1
    x2 = j + k2
    rot_a = (13, 15, 26, 6)
    rot_b = (17, 29, 16, 24)

    def group(x1, x2, rots):
        for r in rots:
            x1 = x1 + x2
            x2 = _rotl(x2, r)
            x2 = x1 ^ x2
        return x1, x2

    x1, x2 = group(x1, x2, rot_a)
    x1, x2 = x1 + k2, x2 + ks2 + jnp.uint32(1)
    x1, x2 = group(x1, x2, rot_b)
    x1, x2 = x1 + ks2, x2 + k1 + jnp.uint32(2)
    x1, x2 = group(x1, x2, rot_a)
    x1, x2 = x1 + k1, x2 + k2 + jnp.uint32(3)
    x1, x2 = group(x1, x2, rot_b)
    x1, x2 = x1 + k2, x2 + ks2 + jnp.uint32(4)
    x1, x2 = group(x1, x2, rot_a)
    x1, x2 = x1 + ks2, x2 + k1 + jnp.uint32(5)
    bits = x1 ^ x2
    fb = (bits >> jnp.uint32(9)) | jnp.uint32(0x3F800000)
    f = lax.bitcast_convert_type(fb, jnp.float32) - jnp.float32(1.0)
    tiny = jnp.float32(np.finfo(np.float32).tiny)
    u = jnp.maximum(tiny, f * (jnp.float32(1.0) - tiny) + tiny)
    g = -jnp.log(-jnp.log(u))
    s = g + lp_ref[...]
    ik = lax.broadcasted_iota(jnp.int32, (_R2, _K), 1)
    for h in (0, 1):
        sh = s[:, h * _K:(h + 1) * _K]
        m = jnp.max(sh, axis=1, keepdims=True)
        z = jnp.min(jnp.where(sh == m, ik, jnp.int32(_K)),
                    axis=1, keepdims=True)
        out_ref[:, h:h + 1] = z


def _select_body_one(mu_ref, z_ref, omu_ref):
    zrow = z_ref[0]
    acc = mu_ref[0, 0]
    for k in range(1, _K):
        acc = jnp.where(zrow == jnp.int32(k), mu_ref[0, k], acc)
    omu_ref[0] = acc


_NW = 32
_TPW = (_S_SC * 4 * 8) // _NW


def _sc_select(sig_hbm, z_hbm, osg_hbm, stage_v, z_v, out_v):
    wid = lax.axis_index("s") * 2 + lax.axis_index("c")

    def task(i, carry):
        t = wid * _TPW + i
        s = t // 32
        dhi = (t % 32) // 8
        bhi = t % 8
        pltpu.sync_copy(z_hbm.at[pl.ds(s, 1), pl.ds(bhi * 128, 128)], z_v)
        pltpu.sync_copy(
            sig_hbm.at[s, :, pl.ds(dhi * 8, 8), pl.ds(bhi * 128, 128)],
            stage_v)
        for g in range(8):
            zv = z_v[0, pl.ds(g * 16, 16)]
            col = lax.iota(jnp.int32, 16) + jnp.int32(g * 16)
            for dlo in range(8):
                row_d = jnp.full((16,), dlo, jnp.int32)
                out_v[dlo, pl.ds(g * 16, 16)] = plsc.load_gather(
                    stage_v, [zv, row_d, col])
        pltpu.sync_copy(
            out_v, osg_hbm.at[s, pl.ds(dhi * 8, 8), pl.ds(bhi * 128, 128)])
        return carry

    lax.fori_loop(0, _TPW, task, 0)


def _sc_select_call():
    return functools.partial(
        pl.kernel,
        out_type=jax.ShapeDtypeStruct((_S_SC, _D, _B), jnp.float32),
        mesh=plsc.VectorSubcoreMesh(core_axis_name="c", subcore_axis_name="s"),
        scratch_types=[pltpu.VMEM((_K, 8, 128), jnp.float32),
                       pltpu.VMEM((1, 128), jnp.int32),
                       pltpu.VMEM((8, 128), jnp.float32)],
        compiler_params=pltpu.CompilerParams(needs_layout_passes=False),
    )


def kernel(mus, sigmas, pi):
    zkey = jax.random.fold_in(jax.random.key(0), 123)
    kd = jax.random.key_data(zkey).astype(jnp.uint32).reshape(1, 2)
    lp_block = jnp.tile(jnp.log(pi), (_R // _S, 1)).reshape(_R // 2, 2 * _K)

    z_flat = pl.pallas_call(
        _sample_body,
        grid=(_ROWS // _R,),
        in_specs=[
            pl.BlockSpec((1, 2), lambda i: (0, 0)),
            pl.BlockSpec((_R // 2, 2 * _K), lambda i: (0, 0)),
        ],
        out_specs=pl.BlockSpec((_R // 2, 2), lambda i: (i, 0)),
        out_shape=jax.ShapeDtypeStruct((_ROWS // 2, 2), jnp.int32),
    )(kd, lp_block)
    z_sb = z_flat.reshape(_B, _S).T.reshape(_S, 1, _B)

    mus_t = mus.transpose(1, 2, 3, 0)
    sig_t = sigmas.transpose(1, 2, 3, 0)

    osg_lo = _sc_select_call()(_sc_select)(sig_t, z_sb.reshape(_S, _B))

    omu_t = pl.pallas_call(
        _select_body_one,
        grid=(_S, _B // _BB),
        in_specs=[
            pl.BlockSpec((1, _K, _D, _BB), lambda s, b: (s, 0, 0, b)),
            pl.BlockSpec((1, 1, _BB), lambda s, b: (s, 0, b)),
        ],
        out_specs=pl.BlockSpec((1, _D, _BB), lambda s, b: (s, 0, b)),
        out_shape=jax.ShapeDtypeStruct((_S, _D, _B), jnp.float32),
    )(mus_t, z_sb)

    osg_hi = pl.pallas_call(
        _select_body_one,
        grid=(_S_TC, _B // _BB),
        in_specs=[
            pl.BlockSpec((1, _K, _D, _BB), lambda s, b: (s + _S_SC, 0, 0, b)),
            pl.BlockSpec((1, 1, _BB), lambda s, b: (s + _S_SC, 0, b)),
        ],
        out_specs=pl.BlockSpec((1, _D, _BB), lambda s, b: (s, 0, b)),
        out_shape=jax.ShapeDtypeStruct((_S_TC, _D, _B), jnp.float32),
    )(sig_t, z_sb)

    osg_t = jnp.concatenate([osg_lo, osg_hi], axis=0)
    return omu_t.transpose(2, 0, 1), osg_t.transpose(2, 0, 1)


# speedup vs baseline: 1.2452x; 1.0209x over previous
import functools

import jax
import jax.numpy as jnp
import numpy as np
from jax import lax
from jax.experimental import pallas as pl
from jax.experimental.pallas import tpu as pltpu
from jax.experimental.pallas import tpu_sc as plsc

_B, _S, _K, _D = 1024, 16, 64, 32
_ROWS = _B * _S
_R = 2048
_BB = 512
_S_SC = 13
_S_TC = _S - _S_SC


def _rotl(x, d):
    return (x << jnp.uint32(d)) | (x >> jnp.uint32(32 - d))


def _sample_body(key_ref, lp_ref, out_ref):
    k1 = key_ref[0, 0]
    k2 = key_ref[0, 1]
    ks2 = k1 ^ k2 ^ jnp.uint32(0x1BD11BDA)
    _R2, _K2 = _R // 2, 2 * _K
    j0 = jnp.uint32(pl.program_id(0) * (_R * _K))
    jrow = lax.broadcasted_iota(jnp.uint32, (_R2, _K2), 0)
    jcol = lax.broadcasted_iota(jnp.uint32, (_R2, _K2), 1)
    j = j0 + jrow * jnp.uint32(_K2) + jcol
    x1 = jnp.zeros((_R2, _K2), jnp.uint32) + k1
    x2 = j + k2
    rot_a = (13, 15, 26, 6)
    rot_b = (17, 29, 16, 24)

    def group(x1, x2, rots):
        for r in rots:
            x1 = x1 + x2
            x2 = _rotl(x2, r)
            x2 = x1 ^ x2
        return x1, x2

    x1, x2 = group(x1, x2, rot_a)
    x1, x2 = x1 + k2, x2 + ks2 + jnp.uint32(1)
    x1, x2 = group(x1, x2, rot_b)
    x1, x2 = x1 + ks2, x2 + k1 + jnp.uint32(2)
    x1, x2 = group(x1, x2, rot_a)
    x1, x2 = x1 + k1, x2 + k2 + jnp.uint32(3)
    x1, x2 = group(x1, x2, rot_b)
    x1, x2 = x1 + k2, x2 + ks2 + jnp.uint32(4)
    x1, x2 = group(x1, x2, rot_a)
    x1, x2 = x1 + ks2, x2 + k1 + jnp.uint32(5)
    bits = x1 ^ x2
    fb = (bits >> jnp.uint32(9)) | jnp.uint32(0x3F800000)
    f = lax.bitcast_convert_type(fb, jnp.float32) - jnp.float32(1.0)
    tiny = jnp.float32(np.finfo(np.float32).tiny)
    u = jnp.maximum(tiny, f * (jnp.float32(1.0) - tiny) + tiny)
    g = -jnp.log(-jnp.log(u))
    s = g + lp_ref[...]
    ik = lax.broadcasted_iota(jnp.int32, (_R2, _K), 1)
    for h in (0, 1):
        sh = s[:, h * _K:(h + 1) * _K]
        m = jnp.max(sh, axis=1, keepdims=True)
        z = jnp.min(jnp.where(sh == m, ik, jnp.int32(_K)),
                    axis=1, keepdims=True)
        out_ref[:, h:h + 1] = z


def _select_body_one(mu_ref, z_ref, omu_ref):
    zrow = z_ref[0]
    acc = mu_ref[0, 0]
    for k in range(1, _K):
        acc = jnp.where(zrow == jnp.int32(k), mu_ref[0, k], acc)
    omu_ref[0] = acc


_NW = 32
_TPW = (_S_SC * 4 * 8) // _NW


def _sc_select(sig_hbm, z_hbm, osg_hbm, stage_v, z_v, out_v):
    wid = lax.axis_index("s") * 2 + lax.axis_index("c")

    def task(i, carry):
        t = wid * _TPW + i
        s = t // 32
        dhi = (t % 32) // 8
        bhi = t % 8
        pltpu.sync_copy(z_hbm.at[pl.ds(s, 1), pl.ds(bhi * 128, 128)], z_v)
        pltpu.sync_copy(
            sig_hbm.at[s, :, pl.ds(dhi * 8, 8), pl.ds(bhi * 128, 128)],
            stage_v)
        for g in range(8):
            zv = z_v[0, pl.ds(g * 16, 16)]
            col = lax.iota(jnp.int32, 16) + jnp.int32(g * 16)
            for dlo in range(8):
                row_d = jnp.full((16,), dlo, jnp.int32)
                out_v[dlo, pl.ds(g * 16, 16)] = plsc.load_gather(
                    stage_v, [zv, row_d, col])
        pltpu.sync_copy(
            out_v, osg_hbm.at[s, pl.ds(dhi * 8, 8), pl.ds(bhi * 128, 128)])
        return carry

    lax.fori_loop(0, _TPW, task, 0)


def _sc_select_call():
    return functools.partial(
        pl.kernel,
        out_type=jax.ShapeDtypeStruct((_S_SC, _D, _B), jnp.float32),
        mesh=plsc.VectorSubcoreMesh(core_axis_name="c", subcore_axis_name="s"),
        scratch_types=[pltpu.VMEM((_K, 8, 128), jnp.float32),
                       pltpu.VMEM((1, 128), jnp.int32),
                       pltpu.VMEM((8, 128), jnp.float32)],
        compiler_params=pltpu.CompilerParams(needs_layout_passes=False),
    )


def kernel(mus, sigmas, pi):
    zkey = jax.random.fold_in(jax.random.key(0), 123)
    kd = jax.random.key_data(zkey).astype(jnp.uint32).reshape(1, 2)
    lp_block = jnp.tile(jnp.log(pi), (_R // _S, 1)).reshape(_R // 2, 2 * _K)

    z_flat = pl.pallas_call(
        _sample_body,
        grid=(_ROWS // _R,),
        in_specs=[
            pl.BlockSpec((1, 2), lambda i: (0, 0)),
            pl.BlockSpec((_R // 2, 2 * _K), lambda i: (0, 0)),
        ],
        out_specs=pl.BlockSpec((_R // 2, 2), lambda i: (i, 0)),
        out_shape=jax.ShapeDtypeStruct((_ROWS // 2, 2), jnp.int32),
    )(kd, lp_block)
    z_sb = z_flat.reshape(_B, _S).T.reshape(_S, 1, _B)

    mus_t = mus.transpose(1, 2, 3, 0)
    sig_t = sigmas.transpose(1, 2, 3, 0)

    osg_lo = _sc_select_call()(_sc_select)(sig_t, z_sb.reshape(_S, _B))

    omu_t = pl.pallas_call(
        _select_body_one,
        grid=(_S, _B // _BB),
        in_specs=[
            pl.BlockSpec((1, _K, _D, _BB), lambda s, b: (s, 0, 0, b)),
            pl.BlockSpec((1, 1, _BB), lambda s, b: (s, 0, b)),
        ],
        out_specs=pl.BlockSpec((1, _D, _BB), lambda s, b: (s, 0, b)),
        out_shape=jax.ShapeDtypeStruct((_S, _D, _B), jnp.float32),
    )(mus_t, z_sb)

    osg_hi = pl.pallas_call(
        _select_body_one,
        grid=(_S_TC, _B // _BB),
        in_specs=[
            pl.BlockSpec((1, _K, _D, _BB), lambda s, b: (s + _S_SC, 0, 0, b)),
            pl.BlockSpec((1, 1, _BB), lambda s, b: (s + _S_SC, 0, b)),
        ],
        out_specs=pl.BlockSpec((1, _D, _BB), lambda s, b: (s, 0, b)),
        out_shape=jax.ShapeDtypeStruct((_S_TC, _D, _B), jnp.float32),
    )(sig_t, z_sb)

    osg_t = jnp.concatenate([osg_lo, osg_hi], axis=0)
    return omu_t.transpose(2, 0, 1), osg_t.transpose(2, 0, 1)
